# baseline (speedup 1.0000x reference)
import functools

import jax
import jax.numpy as jnp
from jax import lax
from jax.experimental import pallas as pl
from jax.experimental.pallas import tpu as pltpu
from jax.experimental.pallas import tpu_sc as plsc

_NUM_CLASSES = 19
_NWORKERS = 32


def _sc_segment_sums(masks, outputs, row_base):
  B, C, H, W = outputs.shape
  workers_per_batch = _NWORKERS // B
  rows_per_worker = (H - row_base) // workers_per_batch
  row_blocks = rows_per_worker // 8
  col_blocks = W // 128
  n_steps = row_blocks * col_blocks
  assert rows_per_worker % 8 == 0 and W % 128 == 0 and n_steps % 2 == 0

  mesh = plsc.VectorSubcoreMesh(core_axis_name="c", subcore_axis_name="s")
  acc_words = C * (C + 1) * 16

  @functools.partial(
      pl.kernel,
      mesh=mesh,
      out_type=jax.ShapeDtypeStruct((_NWORKERS, acc_words), jnp.float32),
      scratch_types=[
          pltpu.VMEM((2, 8, 128), jnp.int32),
          pltpu.VMEM((2, C, 8, 128), jnp.float32),
          pltpu.VMEM((acc_words,), jnp.float32),
          pltpu.SemaphoreType.DMA,
          pltpu.SemaphoreType.DMA,
          pltpu.SemaphoreType.DMA,
          pltpu.SemaphoreType.DMA,
      ],
      compiler_params=pltpu.CompilerParams(
          needs_layout_passes=False, use_tc_tiling_on_sc=True),
  )
  def sc_k(mask_hbm, out_hbm, part_hbm, mtile, otile, acc,
           sm0, sm1, so0, so1):
    wid = lax.axis_index("s") * 2 + lax.axis_index("c")
    b = wid // workers_per_batch
    q = wid % workers_per_batch
    sems = ((sm0, so0), (sm1, so1))

    zero16 = jnp.zeros((16,), jnp.float32)

    def zrow(i, carry):
      acc[pl.ds(pl.multiple_of(i * 16, 16), 16)] = zero16
      return carry

    lax.fori_loop(0, acc_words // 16, zrow, 0)

    lane = lax.iota(jnp.int32, 16)
    ones = jnp.ones((16,), jnp.float32)
    row_stride = (C + 1) * 16

    def step_slices(t):
      rb = t // col_blocks
      cb = t % col_blocks
      row0 = row_base + q * rows_per_worker + rb * 8
      col0 = cb * 128
      return row0, col0

    def issue(t, k):
      row0, col0 = step_slices(t)
      pltpu.async_copy(
          mask_hbm.at[b, pl.ds(row0, 8), pl.ds(col0, 128)], mtile.at[k],
          sems[k][0])
      pltpu.async_copy(
          out_hbm.at[b, :, pl.ds(row0, 8), pl.ds(col0, 128)], otile.at[k],
          sems[k][1])

    def drain(k):
      pltpu.make_async_copy(
          mask_hbm.at[0, pl.ds(0, 8), pl.ds(0, 128)], mtile.at[k],
          sems[k][0]).wait()
      pltpu.make_async_copy(
          out_hbm.at[0, :, pl.ds(0, 8), pl.ds(0, 128)], otile.at[k],
          sems[k][1]).wait()

    def compute(k):
      @plsc.parallel_loop(0, 64, step=1, unroll=2)
      def group(g):
        rr = g // 8
        cc = pl.multiple_of((g % 8) * 16, 16)
        m = mtile[k, rr, pl.ds(cc, 16)]
        base = m * row_stride + lane
        xs = [otile[k, c, rr, pl.ds(cc, 16)] for c in range(C)]
        plsc.addupdate_scatter(acc, [base + C * 16], ones)
        for c in range(C):
          plsc.addupdate_scatter(acc, [base + c * 16], xs[c])

    issue(0, 0)
    issue(1, 1)

    def step_pair(i, carry):
      t0 = 2 * i
      drain(0)
      compute(0)

      @pl.when(t0 + 2 < n_steps)
      def _():
        issue(t0 + 2, 0)

      drain(1)
      compute(1)

      @pl.when(t0 + 3 < n_steps)
      def _():
        issue(t0 + 3, 1)

      return carry

    lax.fori_loop(0, n_steps // 2, step_pair, 0)
    pltpu.sync_copy(acc, part_hbm.at[wid])

  return sc_k(masks, outputs)


_TC_ROWS = 128


def _tc_partial(masks, outputs, ht):
  B, C, H, W = outputs.shape
  blk = 8
  nrb = ht // blk

  def body(mask_ref, out_ref, acc_ref):
    rb = pl.program_id(1)

    @pl.when(rb == 0)
    def _():
      acc_ref[...] = jnp.zeros_like(acc_ref)

    msk = mask_ref[0]
    x = out_ref[0]
    iota_c = lax.broadcasted_iota(jnp.int32, (C, W), 0)
    ones2 = jnp.ones((1, W), jnp.float32)
    dn_nt = (((1,), (1,)), ((), ()))
    ps = jnp.zeros((C, C), jnp.float32)
    cnt = jnp.zeros((1, C), jnp.float32)
    for r in range(blk):
      ohr = (msk[r][None, :] == iota_c).astype(jnp.float32)
      xr = x[:, r, :]
      ps = ps + lax.dot_general(xr, ohr, dn_nt,
                                preferred_element_type=jnp.float32)
      cnt = cnt + lax.dot_general(ones2, ohr, dn_nt,
                                  preferred_element_type=jnp.float32)
    acc_ref[0, :C, :] += ps
    acc_ref[0, C:, :] += cnt

  return pl.pallas_call(
      body,
      grid=(B, nrb),
      in_specs=[
          pl.BlockSpec((1, blk, W), lambda b, rb: (b, rb, 0)),
          pl.BlockSpec((1, C, blk, W), lambda b, rb: (b, 0, rb, 0)),
      ],
      out_specs=pl.BlockSpec((1, C + 1, C), lambda b, rb: (b, 0, 0)),
      out_shape=jax.ShapeDtypeStruct((B, C + 1, C), jnp.float32),
  )(masks, outputs)


def _finalize_body(part_ref, tc_ref, out_ref):
  C = _NUM_CLASSES
  x = part_ref[...]
  B = x.shape[0] // 4
  x = x.reshape(x.shape[0], C, C + 1, 16)
  r = jnp.sum(jnp.sum(x, axis=3).reshape(B, 4, C, C + 1), axis=1)
  xt = tc_ref[...]
  protosum = r[:, :, :C] + jnp.swapaxes(xt[:, :C, :], 1, 2)
  counts = r[:, :, C] + xt[:, C, :]
  safe = jnp.maximum(counts, 1.0)
  proto = protosum / safe[:, :, None]
  mx = jnp.max(proto, axis=-1, keepdims=True)
  sh = proto - mx
  lse = jnp.log(jnp.sum(jnp.exp(sh), axis=-1, keepdims=True))
  logp = sh - lse
  i = lax.broadcasted_iota(jnp.int32, (C, C), 0)
  j = lax.broadcasted_iota(jnp.int32, (C, C), 1)
  smooth = jnp.where(i == j, 0.9, 0.1 / 8.0)
  row_loss = jnp.sum(smooth[None, :, :] * logp, axis=-1)
  class_ids = lax.broadcasted_iota(jnp.int32, counts.shape, 1)
  present = counts > 0.0
  min_present = jnp.min(jnp.where(present, class_ids, C), axis=1,
                        keepdims=True)
  w = (present & (class_ids != min_present)).astype(jnp.float32)
  loss = -jnp.sum(w * row_loss) / jnp.sum(w)
  out_ref[...] = jnp.broadcast_to(loss, (1, 1))


def kernel(masks, outputs):
  B, C, H, W = outputs.shape
  masks = masks.astype(jnp.int32)
  partials = _sc_segment_sums(masks, outputs, _TC_ROWS)
  tc_part = _tc_partial(masks, outputs, _TC_ROWS)
  loss = pl.pallas_call(
      _finalize_body,
      out_shape=jax.ShapeDtypeStruct((1, 1), jnp.float32),
  )(partials, tc_part)
  return loss[0, 0]


# speedup vs baseline: 1.4041x; 1.4041x over previous
import functools

import jax
import jax.numpy as jnp
from jax import lax
from jax.experimental import pallas as pl
from jax.experimental.pallas import tpu as pltpu
from jax.experimental.pallas import tpu_sc as plsc

_NUM_CLASSES = 19
_NWORKERS = 32


def _sc_segment_sums(masks, outputs, row_base):
  B, C, H, W = outputs.shape
  workers_per_batch = _NWORKERS // B
  rows_per_worker = (H - row_base) // workers_per_batch
  row_blocks = rows_per_worker // 8
  col_blocks = W // 128
  n_steps = row_blocks * col_blocks
  assert rows_per_worker % 8 == 0 and W % 128 == 0 and n_steps % 2 == 0

  mesh = plsc.VectorSubcoreMesh(core_axis_name="c", subcore_axis_name="s")
  acc_words = C * (C + 1) * 16

  @functools.partial(
      pl.kernel,
      mesh=mesh,
      out_type=jax.ShapeDtypeStruct((_NWORKERS, acc_words), jnp.float32),
      scratch_types=[
          pltpu.VMEM((2, 8, 128), jnp.int32),
          pltpu.VMEM((2, C, 8, 128), jnp.float32),
          pltpu.VMEM((acc_words,), jnp.float32),
          pltpu.SemaphoreType.DMA,
          pltpu.SemaphoreType.DMA,
          pltpu.SemaphoreType.DMA,
          pltpu.SemaphoreType.DMA,
      ],
      compiler_params=pltpu.CompilerParams(
          needs_layout_passes=False, use_tc_tiling_on_sc=True),
  )
  def sc_k(mask_hbm, out_hbm, part_hbm, mtile, otile, acc,
           sm0, sm1, so0, so1):
    wid = lax.axis_index("s") * 2 + lax.axis_index("c")
    b = wid // workers_per_batch
    q = wid % workers_per_batch
    sems = ((sm0, so0), (sm1, so1))

    zero16 = jnp.zeros((16,), jnp.float32)

    def zrow(i, carry):
      acc[pl.ds(pl.multiple_of(i * 16, 16), 16)] = zero16
      return carry

    lax.fori_loop(0, acc_words // 16, zrow, 0)

    lane = lax.iota(jnp.int32, 16)
    ones = jnp.ones((16,), jnp.float32)
    row_stride = (C + 1) * 16

    def step_slices(t):
      rb = t // col_blocks
      cb = t % col_blocks
      row0 = row_base + q * rows_per_worker + rb * 8
      col0 = cb * 128
      return row0, col0

    def issue(t, k):
      row0, col0 = step_slices(t)
      pltpu.async_copy(
          mask_hbm.at[b, pl.ds(row0, 8), pl.ds(col0, 128)], mtile.at[k],
          sems[k][0])
      pltpu.async_copy(
          out_hbm.at[b, :, pl.ds(row0, 8), pl.ds(col0, 128)], otile.at[k],
          sems[k][1])

    def drain(k):
      pltpu.make_async_copy(
          mask_hbm.at[0, pl.ds(0, 8), pl.ds(0, 128)], mtile.at[k],
          sems[k][0]).wait()
      pltpu.make_async_copy(
          out_hbm.at[0, :, pl.ds(0, 8), pl.ds(0, 128)], otile.at[k],
          sems[k][1]).wait()

    def compute(k):
      @plsc.parallel_loop(0, 64, step=1, unroll=2)
      def group(g):
        rr = g // 8
        cc = pl.multiple_of((g % 8) * 16, 16)
        m = mtile[k, rr, pl.ds(cc, 16)]
        base = m * row_stride + lane
        xs = [otile[k, c, rr, pl.ds(cc, 16)] for c in range(C)]
        plsc.addupdate_scatter(acc, [base + C * 16], ones)
        for c in range(C):
          plsc.addupdate_scatter(acc, [base + c * 16], xs[c])

    issue(0, 0)
    issue(1, 1)

    def step_pair(i, carry):
      t0 = 2 * i
      drain(0)
      compute(0)

      @pl.when(t0 + 2 < n_steps)
      def _():
        issue(t0 + 2, 0)

      drain(1)
      compute(1)

      @pl.when(t0 + 3 < n_steps)
      def _():
        issue(t0 + 3, 1)

      return carry

    lax.fori_loop(0, n_steps // 2, step_pair, 0)
    pltpu.sync_copy(acc, part_hbm.at[wid])

  return sc_k(masks, outputs)


_TC_ROWS = 64


def _tc_partial(masks, outputs, ht):
  B, C, H, W = outputs.shape
  blk = 8
  nrb = ht // blk

  def body(mask_ref, out_ref, acc_ref):
    rb = pl.program_id(1)

    @pl.when(rb == 0)
    def _():
      acc_ref[...] = jnp.zeros_like(acc_ref)

    msk = mask_ref[0]
    x = out_ref[0]
    iota_c = lax.broadcasted_iota(jnp.int32, (C, W), 0)
    ones2 = jnp.ones((1, W), jnp.float32)
    dn_nt = (((1,), (1,)), ((), ()))
    ps = jnp.zeros((C, C), jnp.float32)
    cnt = jnp.zeros((1, C), jnp.float32)
    for r in range(blk):
      ohr = (msk[r][None, :] == iota_c).astype(jnp.float32)
      xr = x[:, r, :]
      ps = ps + lax.dot_general(xr, ohr, dn_nt,
                                preferred_element_type=jnp.float32)
      cnt = cnt + lax.dot_general(ones2, ohr, dn_nt,
                                  preferred_element_type=jnp.float32)
    acc_ref[0, :C, :] += ps
    acc_ref[0, C:, :] += cnt

  return pl.pallas_call(
      body,
      grid=(B, nrb),
      in_specs=[
          pl.BlockSpec((1, blk, W), lambda b, rb: (b, rb, 0)),
          pl.BlockSpec((1, C, blk, W), lambda b, rb: (b, 0, rb, 0)),
      ],
      out_specs=pl.BlockSpec((1, C + 1, C), lambda b, rb: (b, 0, 0)),
      out_shape=jax.ShapeDtypeStruct((B, C + 1, C), jnp.float32),
  )(masks, outputs)


def _finalize_body(part_ref, tc_ref, out_ref):
  C = _NUM_CLASSES
  x = part_ref[...]
  B = x.shape[0] // 4
  x = x.reshape(x.shape[0], C, C + 1, 16)
  r = jnp.sum(jnp.sum(x, axis=3).reshape(B, 4, C, C + 1), axis=1)
  xt = tc_ref[...]
  protosum = r[:, :, :C] + jnp.swapaxes(xt[:, :C, :], 1, 2)
  counts = r[:, :, C] + xt[:, C, :]
  safe = jnp.maximum(counts, 1.0)
  proto = protosum / safe[:, :, None]
  mx = jnp.max(proto, axis=-1, keepdims=True)
  sh = proto - mx
  lse = jnp.log(jnp.sum(jnp.exp(sh), axis=-1, keepdims=True))
  logp = sh - lse
  i = lax.broadcasted_iota(jnp.int32, (C, C), 0)
  j = lax.broadcasted_iota(jnp.int32, (C, C), 1)
  smooth = jnp.where(i == j, 0.9, 0.1 / 8.0)
  row_loss = jnp.sum(smooth[None, :, :] * logp, axis=-1)
  class_ids = lax.broadcasted_iota(jnp.int32, counts.shape, 1)
  present = counts > 0.0
  min_present = jnp.min(jnp.where(present, class_ids, C), axis=1,
                        keepdims=True)
  w = (present & (class_ids != min_present)).astype(jnp.float32)
  loss = -jnp.sum(w * row_loss) / jnp.sum(w)
  out_ref[...] = jnp.broadcast_to(loss, (1, 1))


def kernel(masks, outputs):
  B, C, H, W = outputs.shape
  masks = masks.astype(jnp.int32)
  partials = _sc_segment_sums(masks, outputs, _TC_ROWS)
  tc_part = _tc_partial(masks, outputs, _TC_ROWS)
  loss = pl.pallas_call(
      _finalize_body,
      out_shape=jax.ShapeDtypeStruct((1, 1), jnp.float32),
  )(partials, tc_part)
  return loss[0, 0]


# speedup vs baseline: 1.4521x; 1.0342x over previous
import functools

import jax
import jax.numpy as jnp
from jax import lax
from jax.experimental import pallas as pl
from jax.experimental.pallas import tpu as pltpu
from jax.experimental.pallas import tpu_sc as plsc

_NUM_CLASSES = 19
_NWORKERS = 32


def _sc_segment_sums(masks, outputs, row_base):
  B, C, H, W = outputs.shape
  workers_per_batch = _NWORKERS // B
  rows_per_worker = (H - row_base) // workers_per_batch
  row_blocks = rows_per_worker // 8
  col_blocks = W // 128
  n_steps = row_blocks * col_blocks
  assert rows_per_worker % 8 == 0 and W % 128 == 0 and n_steps % 2 == 0

  mesh = plsc.VectorSubcoreMesh(core_axis_name="c", subcore_axis_name="s")
  acc_rows = 384
  acc_words = acc_rows * 16

  @functools.partial(
      pl.kernel,
      mesh=mesh,
      out_type=jax.ShapeDtypeStruct((_NWORKERS, acc_rows), jnp.float32),
      scratch_types=[
          pltpu.VMEM((2, 8, 128), jnp.int32),
          pltpu.VMEM((2, C, 8, 128), jnp.float32),
          pltpu.VMEM((acc_words,), jnp.float32),
          pltpu.VMEM((acc_rows,), jnp.float32),
          pltpu.SemaphoreType.DMA,
          pltpu.SemaphoreType.DMA,
          pltpu.SemaphoreType.DMA,
          pltpu.SemaphoreType.DMA,
      ],
      compiler_params=pltpu.CompilerParams(
          needs_layout_passes=False, use_tc_tiling_on_sc=True),
  )
  def sc_k(mask_hbm, out_hbm, part_hbm, mtile, otile, acc, acc2,
           sm0, sm1, so0, so1):
    wid = lax.axis_index("s") * 2 + lax.axis_index("c")
    b = wid // workers_per_batch
    q = wid % workers_per_batch
    sems = ((sm0, so0), (sm1, so1))

    zero16 = jnp.zeros((16,), jnp.float32)

    def zrow(i, carry):
      acc[pl.ds(pl.multiple_of(i * 16, 16), 16)] = zero16
      return carry

    lax.fori_loop(0, acc_words // 16, zrow, 0)

    lane = lax.iota(jnp.int32, 16)
    ones = jnp.ones((16,), jnp.float32)
    row_stride = (C + 1) * 16

    def step_slices(t):
      rb = t // col_blocks
      cb = t % col_blocks
      row0 = row_base + q * rows_per_worker + rb * 8
      col0 = cb * 128
      return row0, col0

    def issue(t, k):
      row0, col0 = step_slices(t)
      pltpu.async_copy(
          mask_hbm.at[b, pl.ds(row0, 8), pl.ds(col0, 128)], mtile.at[k],
          sems[k][0])
      pltpu.async_copy(
          out_hbm.at[b, :, pl.ds(row0, 8), pl.ds(col0, 128)], otile.at[k],
          sems[k][1])

    def drain(k):
      pltpu.make_async_copy(
          mask_hbm.at[0, pl.ds(0, 8), pl.ds(0, 128)], mtile.at[k],
          sems[k][0]).wait()
      pltpu.make_async_copy(
          out_hbm.at[0, :, pl.ds(0, 8), pl.ds(0, 128)], otile.at[k],
          sems[k][1]).wait()

    def compute(k):
      @plsc.parallel_loop(0, 64, step=1, unroll=2)
      def group(g):
        rr = g // 8
        cc = pl.multiple_of((g % 8) * 16, 16)
        m = mtile[k, rr, pl.ds(cc, 16)]
        base = m * row_stride + lane
        xs = [otile[k, c, rr, pl.ds(cc, 16)] for c in range(C)]
        plsc.addupdate_scatter(acc, [base + C * 16], ones)
        for c in range(C):
          plsc.addupdate_scatter(acc, [base + c * 16], xs[c])

    issue(0, 0)
    issue(1, 1)

    def step_pair(i, carry):
      t0 = 2 * i
      drain(0)
      compute(0)

      @pl.when(t0 + 2 < n_steps)
      def _():
        issue(t0 + 2, 0)

      drain(1)
      compute(1)

      @pl.when(t0 + 3 < n_steps)
      def _():
        issue(t0 + 3, 1)

      return carry

    lax.fori_loop(0, n_steps // 2, step_pair, 0)

    row16 = lane * 16

    def redblk(jb, carry):
      base = jb * 256
      s = jnp.zeros((16,), jnp.float32)
      for l in range(16):
        s = s + plsc.load_gather(acc, [row16 + (base + l)])
      acc2[pl.ds(pl.multiple_of(jb * 16, 16), 16)] = s
      return carry

    lax.fori_loop(0, acc_rows // 16, redblk, 0)
    pltpu.sync_copy(acc2, part_hbm.at[wid])

  return sc_k(masks, outputs)


_TC_ROWS = 64


def _tc_partial(masks, outputs, ht):
  B, C, H, W = outputs.shape
  blk = 8
  nrb = ht // blk

  def body(mask_ref, out_ref, acc_ref):
    rb = pl.program_id(1)

    @pl.when(rb == 0)
    def _():
      acc_ref[...] = jnp.zeros_like(acc_ref)

    msk = mask_ref[0]
    x = out_ref[0]
    iota_c = lax.broadcasted_iota(jnp.int32, (C, W), 0)
    ones2 = jnp.ones((1, W), jnp.float32)
    dn_nt = (((1,), (1,)), ((), ()))
    ps = jnp.zeros((C, C), jnp.float32)
    cnt = jnp.zeros((1, C), jnp.float32)
    for r in range(blk):
      ohr = (msk[r][None, :] == iota_c).astype(jnp.float32)
      xr = x[:, r, :]
      ps = ps + lax.dot_general(xr, ohr, dn_nt,
                                preferred_element_type=jnp.float32)
      cnt = cnt + lax.dot_general(ones2, ohr, dn_nt,
                                  preferred_element_type=jnp.float32)
    acc_ref[0, :C, :] += ps
    acc_ref[0, C:, :] += cnt

  return pl.pallas_call(
      body,
      grid=(B, nrb),
      in_specs=[
          pl.BlockSpec((1, blk, W), lambda b, rb: (b, rb, 0)),
          pl.BlockSpec((1, C, blk, W), lambda b, rb: (b, 0, rb, 0)),
      ],
      out_specs=pl.BlockSpec((1, C + 1, C), lambda b, rb: (b, 0, 0)),
      out_shape=jax.ShapeDtypeStruct((B, C + 1, C), jnp.float32),
  )(masks, outputs)


def _finalize_body(part_ref, tc_ref, out_ref):
  C = _NUM_CLASSES
  x = part_ref[...]
  B = x.shape[0] // 4
  x = x[:, :C * (C + 1)].reshape(B, 4, C, C + 1)
  r = jnp.sum(x, axis=1)
  xt = tc_ref[...]
  protosum = r[:, :, :C] + jnp.swapaxes(xt[:, :C, :], 1, 2)
  counts = r[:, :, C] + xt[:, C, :]
  safe = jnp.maximum(counts, 1.0)
  proto = protosum / safe[:, :, None]
  mx = jnp.max(proto, axis=-1, keepdims=True)
  sh = proto - mx
  lse = jnp.log(jnp.sum(jnp.exp(sh), axis=-1, keepdims=True))
  logp = sh - lse
  i = lax.broadcasted_iota(jnp.int32, (C, C), 0)
  j = lax.broadcasted_iota(jnp.int32, (C, C), 1)
  smooth = jnp.where(i == j, 0.9, 0.1 / 8.0)
  row_loss = jnp.sum(smooth[None, :, :] * logp, axis=-1)
  class_ids = lax.broadcasted_iota(jnp.int32, counts.shape, 1)
  present = counts > 0.0
  min_present = jnp.min(jnp.where(present, class_ids, C), axis=1,
                        keepdims=True)
  w = (present & (class_ids != min_present)).astype(jnp.float32)
  loss = -jnp.sum(w * row_loss) / jnp.sum(w)
  out_ref[...] = jnp.broadcast_to(loss, (1, 1))


def kernel(masks, outputs):
  B, C, H, W = outputs.shape
  masks = masks.astype(jnp.int32)
  partials = _sc_segment_sums(masks, outputs, _TC_ROWS)
  tc_part = _tc_partial(masks, outputs, _TC_ROWS)
  loss = pl.pallas_call(
      _finalize_body,
      out_shape=jax.ShapeDtypeStruct((1, 1), jnp.float32),
  )(partials, tc_part)
  return loss[0, 0]
